# Initial kernel scaffold; baseline (speedup 1.0000x reference)
import jax
import jax.numpy as jnp
from jax.experimental import pallas as pl


def kernel(pmf, times, events, time_bins):
    raise NotImplementedError("write your pallas kernel here")


# speedup vs baseline: 9.3794x; 9.3794x over previous
import jax
import jax.numpy as jnp
from jax.experimental import pallas as pl
from jax.experimental.pallas import tpu as pltpu

_ALPHA = 0.5
_SIGMA = 0.1
_EPS = 1e-07


def _deephit_body(pmf_ref, pmfT_ref, tcol_ref, trow_ref, ecol_ref, tbrow_ref,
                  out_ref,
                  cdfT_ref, diag_ref, bin_ref, nll_ref, rank_ref, np_ref,
                  ev_ref):
    nb = pl.num_programs(0)
    pid = pl.program_id(0)
    n, t = pmf_ref.shape
    bi = n // nb

    @pl.when(pid == 0)
    def _stage_a():
        pmf = pmf_ref[...]
        r = jax.lax.broadcasted_iota(jnp.int32, (t, t), 0)
        c = jax.lax.broadcasted_iota(jnp.int32, (t, t), 1)
        upper = (r <= c).astype(jnp.float32)
        lower = (r >= c).astype(jnp.float32)
        cdf = jnp.dot(pmf, upper, preferred_element_type=jnp.float32)
        rev = jnp.dot(pmf, lower, preferred_element_type=jnp.float32)
        cdfT_ref[...] = jnp.dot(lower, pmfT_ref[...],
                                preferred_element_type=jnp.float32)
        tcol = tcol_ref[...]
        tb = tbrow_ref[...]
        cnt_below = jnp.sum((tb < tcol).astype(jnp.float32), axis=1,
                            keepdims=True)
        binf = jnp.clip(cnt_below - 1.0, 0.0, float(t - 1))
        bin_ref[...] = binf
        lane = jax.lax.broadcasted_iota(jnp.int32, (n, t), 1).astype(
            jnp.float32)
        oh = (lane == binf).astype(jnp.float32)
        pmf_at = jnp.sum(oh * pmf, axis=1, keepdims=True)
        surv = jnp.sum(oh * rev, axis=1, keepdims=True)
        diag_ref[...] = jnp.sum(oh * cdf, axis=1, keepdims=True)
        ev = ecol_ref[...]
        nll = jnp.where(ev == 1.0, -jnp.log(pmf_at + _EPS),
                        -jnp.log(surv + _EPS))
        nll_ref[...] = jnp.sum(nll, axis=0, keepdims=True)
        ev_ref[...] = jnp.sum(ev, axis=0, keepdims=True)
        rank_ref[...] = jnp.zeros_like(rank_ref)
        np_ref[...] = jnp.zeros_like(np_ref)

    sl = pl.ds(pid * bi, bi)
    b = bin_ref[sl, :]
    lane = jax.lax.broadcasted_iota(jnp.int32, (bi, t), 1).astype(jnp.float32)
    onehot = (lane == b).astype(jnp.float32)
    m = jnp.dot(onehot, cdfT_ref[...],
                preferred_element_type=jnp.float32)
    d = diag_ref[sl, :]
    tcb = tcol_ref[sl, :]
    later = trow_ref[...] > tcb
    ex = jnp.exp((m - d) * (1.0 / _SIGMA))
    s = jnp.sum(jnp.where(later, ex, 0.0), axis=1, keepdims=True)
    cnt = jnp.sum(later.astype(jnp.float32), axis=1, keepdims=True)
    eb = ecol_ref[sl, :]
    inc = jnp.logical_and(eb == 1.0, cnt > 0.0)
    per = jnp.where(inc, s / jnp.maximum(cnt, 1.0), 0.0)
    rank_ref[...] += jnp.sum(per, axis=0, keepdims=True)
    np_ref[...] += jnp.sum(inc.astype(jnp.float32), axis=0, keepdims=True)

    @pl.when(pid == nb - 1)
    def _finish():
        npv = np_ref[...]
        rk = rank_ref[...]
        evs = ev_ref[...]
        nll_s = nll_ref[...]
        add = jnp.where(jnp.logical_and(evs > 1.0, npv > 0.0),
                        _ALPHA * rk / jnp.maximum(npv, 1.0),
                        jnp.zeros_like(rk))
        out_ref[...] = nll_s / float(n) + add


def kernel(pmf, times, events, time_bins):
    n, t = pmf.shape
    bi = 256
    nb = n // bi
    pmfT = pmf.T
    tcol = times.reshape(n, 1)
    trow = times.reshape(1, n)
    ecol = events.astype(jnp.float32).reshape(n, 1)
    tbrow = time_bins.reshape(1, t)
    out = pl.pallas_call(
        _deephit_body,
        grid=(nb,),
        in_specs=[
            pl.BlockSpec((n, t), lambda k: (0, 0)),
            pl.BlockSpec((t, n), lambda k: (0, 0)),
            pl.BlockSpec((n, 1), lambda k: (0, 0)),
            pl.BlockSpec((1, n), lambda k: (0, 0)),
            pl.BlockSpec((n, 1), lambda k: (0, 0)),
            pl.BlockSpec((1, t), lambda k: (0, 0)),
        ],
        out_specs=pl.BlockSpec((1, 1), lambda k: (0, 0)),
        out_shape=jax.ShapeDtypeStruct((1, 1), jnp.float32),
        scratch_shapes=[
            pltpu.VMEM((t, n), jnp.float32),
            pltpu.VMEM((n, 1), jnp.float32),
            pltpu.VMEM((n, 1), jnp.float32),
            pltpu.VMEM((1, 1), jnp.float32),
            pltpu.VMEM((1, 1), jnp.float32),
            pltpu.VMEM((1, 1), jnp.float32),
            pltpu.VMEM((1, 1), jnp.float32),
        ],
    )(pmf, pmfT, tcol, trow, ecol, tbrow)
    return out[0, 0]


# speedup vs baseline: 11.2105x; 1.1952x over previous
import jax
import jax.numpy as jnp
from jax.experimental import pallas as pl
from jax.experimental.pallas import tpu as pltpu

_ALPHA = 0.5
_SIGMA = 0.1
_EPS = 1e-07


def _deephit_body(pmf_ref, pmfT_ref, tcol_ref, trow_ref, ecol_ref, tbrow_ref,
                  out_ref,
                  w_ref, ed_ref, bin_ref, nll_ref, rank_ref, np_ref,
                  ev_ref):
    nb = pl.num_programs(0)
    pid = pl.program_id(0)
    n, t = pmf_ref.shape
    bi = n // nb

    @pl.when(pid == 0)
    def _stage_a():
        pmf = pmf_ref[...]
        r = jax.lax.broadcasted_iota(jnp.int32, (t, t), 0)
        c = jax.lax.broadcasted_iota(jnp.int32, (t, t), 1)
        upper = (r <= c).astype(jnp.float32)
        lower = (r >= c).astype(jnp.float32)
        cdf = jnp.dot(pmf, upper, preferred_element_type=jnp.float32)
        rev = jnp.dot(pmf, lower, preferred_element_type=jnp.float32)
        cdfT = jnp.dot(lower, pmfT_ref[...],
                       preferred_element_type=jnp.float32)
        w_ref[...] = jnp.exp(cdfT * (1.0 / _SIGMA)).astype(jnp.bfloat16)
        tcol = tcol_ref[...]
        tb = tbrow_ref[...]
        cnt_below = jnp.sum((tb < tcol).astype(jnp.float32), axis=1,
                            keepdims=True)
        binf = jnp.clip(cnt_below - 1.0, 0.0, float(t - 1))
        bin_ref[...] = binf
        lane = jax.lax.broadcasted_iota(jnp.int32, (n, t), 1).astype(
            jnp.float32)
        oh = (lane == binf).astype(jnp.float32)
        pmf_at = jnp.sum(oh * pmf, axis=1, keepdims=True)
        surv = jnp.sum(oh * rev, axis=1, keepdims=True)
        diag = jnp.sum(oh * cdf, axis=1, keepdims=True)
        ed_ref[...] = jnp.exp(diag * (-1.0 / _SIGMA))
        ev = ecol_ref[...]
        nll = jnp.where(ev == 1.0, -jnp.log(pmf_at + _EPS),
                        -jnp.log(surv + _EPS))
        nll_ref[...] = jnp.sum(nll, axis=0, keepdims=True)
        ev_ref[...] = jnp.sum(ev, axis=0, keepdims=True)
        rank_ref[...] = jnp.zeros_like(rank_ref)
        np_ref[...] = jnp.zeros_like(np_ref)

    sl = pl.ds(pid * bi, bi)
    b = bin_ref[sl, :]
    lane = jax.lax.broadcasted_iota(jnp.int32, (bi, t), 1).astype(jnp.float32)
    onehot = (lane == b).astype(jnp.bfloat16)
    m = jnp.dot(onehot, w_ref[...],
                preferred_element_type=jnp.float32)
    ed = ed_ref[sl, :]
    tcb = tcol_ref[sl, :]
    later = trow_ref[...] > tcb
    ex = m * ed
    s = jnp.sum(jnp.where(later, ex, 0.0), axis=1, keepdims=True)
    cnt = jnp.sum(later.astype(jnp.float32), axis=1, keepdims=True)
    eb = ecol_ref[sl, :]
    inc = jnp.logical_and(eb == 1.0, cnt > 0.0)
    per = jnp.where(inc, s / jnp.maximum(cnt, 1.0), 0.0)
    rank_ref[...] += jnp.sum(per, axis=0, keepdims=True)
    np_ref[...] += jnp.sum(inc.astype(jnp.float32), axis=0, keepdims=True)

    @pl.when(pid == nb - 1)
    def _finish():
        npv = np_ref[...]
        rk = rank_ref[...]
        evs = ev_ref[...]
        nll_s = nll_ref[...]
        add = jnp.where(jnp.logical_and(evs > 1.0, npv > 0.0),
                        _ALPHA * rk / jnp.maximum(npv, 1.0),
                        jnp.zeros_like(rk))
        out_ref[...] = nll_s / float(n) + add


def kernel(pmf, times, events, time_bins):
    n, t = pmf.shape
    bi = 256
    nb = n // bi
    pmfT = pmf.T
    tcol = times.reshape(n, 1)
    trow = times.reshape(1, n)
    ecol = events.astype(jnp.float32).reshape(n, 1)
    tbrow = time_bins.reshape(1, t)
    out = pl.pallas_call(
        _deephit_body,
        grid=(nb,),
        in_specs=[
            pl.BlockSpec((n, t), lambda k: (0, 0)),
            pl.BlockSpec((t, n), lambda k: (0, 0)),
            pl.BlockSpec((n, 1), lambda k: (0, 0)),
            pl.BlockSpec((1, n), lambda k: (0, 0)),
            pl.BlockSpec((n, 1), lambda k: (0, 0)),
            pl.BlockSpec((1, t), lambda k: (0, 0)),
        ],
        out_specs=pl.BlockSpec((1, 1), lambda k: (0, 0)),
        out_shape=jax.ShapeDtypeStruct((1, 1), jnp.float32),
        scratch_shapes=[
            pltpu.VMEM((t, n), jnp.bfloat16),
            pltpu.VMEM((n, 1), jnp.float32),
            pltpu.VMEM((n, 1), jnp.float32),
            pltpu.VMEM((1, 1), jnp.float32),
            pltpu.VMEM((1, 1), jnp.float32),
            pltpu.VMEM((1, 1), jnp.float32),
            pltpu.VMEM((1, 1), jnp.float32),
        ],
    )(pmf, pmfT, tcol, trow, ecol, tbrow)
    return out[0, 0]


# speedup vs baseline: 13.1437x; 1.1724x over previous
import jax
import jax.numpy as jnp
from jax.experimental import pallas as pl
from jax.experimental.pallas import tpu as pltpu

_ALPHA = 0.5
_SIGMA = 0.1
_EPS = 1e-07


def _deephit_body(pmf_ref, pmfT_ref, tcol_ref, trow_ref, ecol_ref, tbrow_ref,
                  out_ref,
                  w_ref, ed_ref, bin_ref, nll_ref, rank_ref, np_ref,
                  ev_ref):
    nb = pl.num_programs(0)
    pid = pl.program_id(0)
    n, t = pmf_ref.shape
    bi = n // nb

    @pl.when(pid == 0)
    def _stage_a():
        pmf = pmf_ref[...]
        r = jax.lax.broadcasted_iota(jnp.int32, (t, t), 0)
        c = jax.lax.broadcasted_iota(jnp.int32, (t, t), 1)
        upper = (r <= c).astype(jnp.float32)
        lower = (r >= c).astype(jnp.float32)
        cdf = jnp.dot(pmf, upper, preferred_element_type=jnp.float32)
        rev = jnp.dot(pmf, lower, preferred_element_type=jnp.float32)
        cdfT = jnp.dot(lower, pmfT_ref[...],
                       preferred_element_type=jnp.float32)
        w_ref[...] = jnp.exp(cdfT * (1.0 / _SIGMA)).astype(jnp.bfloat16)
        tcol = tcol_ref[...]
        tb = tbrow_ref[...]
        cnt_below = jnp.sum((tb < tcol).astype(jnp.float32), axis=1,
                            keepdims=True)
        binf = jnp.clip(cnt_below - 1.0, 0.0, float(t - 1))
        bin_ref[...] = binf
        lane = jax.lax.broadcasted_iota(jnp.int32, (n, t), 1).astype(
            jnp.float32)
        oh = (lane == binf).astype(jnp.float32)
        pmf_at = jnp.sum(oh * pmf, axis=1, keepdims=True)
        surv = jnp.sum(oh * rev, axis=1, keepdims=True)
        diag = jnp.sum(oh * cdf, axis=1, keepdims=True)
        ed_ref[...] = jnp.exp(diag * (-1.0 / _SIGMA))
        ev = ecol_ref[...]
        nll = jnp.where(ev == 1.0, -jnp.log(pmf_at + _EPS),
                        -jnp.log(surv + _EPS))
        nll_ref[...] = jnp.sum(nll, axis=0, keepdims=True)
        ev_ref[...] = jnp.sum(ev, axis=0, keepdims=True)
        rank_ref[...] = jnp.zeros_like(rank_ref)
        np_ref[...] = jnp.zeros_like(np_ref)

    sl = pl.ds(pid * bi, bi)
    b = bin_ref[sl, :]
    lane = jax.lax.broadcasted_iota(jnp.int32, (bi, t), 1).astype(jnp.float32)
    onehot = (lane == b).astype(jnp.bfloat16)
    m = jnp.dot(onehot, w_ref[...],
                preferred_element_type=jnp.float32)
    ed = ed_ref[sl, :]
    tcb = tcol_ref[sl, :]
    later = jnp.where(trow_ref[...] > tcb, 1.0, 0.0)
    s = jnp.sum(later * m, axis=1, keepdims=True) * ed
    cnt = jnp.sum(later, axis=1, keepdims=True)
    eb = ecol_ref[sl, :]
    inc = jnp.logical_and(eb == 1.0, cnt > 0.0)
    per = jnp.where(inc, s / jnp.maximum(cnt, 1.0), 0.0)
    rank_ref[...] += jnp.sum(per, axis=0, keepdims=True)
    np_ref[...] += jnp.sum(inc.astype(jnp.float32), axis=0, keepdims=True)

    @pl.when(pid == nb - 1)
    def _finish():
        npv = np_ref[...]
        rk = rank_ref[...]
        evs = ev_ref[...]
        nll_s = nll_ref[...]
        add = jnp.where(jnp.logical_and(evs > 1.0, npv > 0.0),
                        _ALPHA * rk / jnp.maximum(npv, 1.0),
                        jnp.zeros_like(rk))
        out_ref[...] = nll_s / float(n) + add


def kernel(pmf, times, events, time_bins):
    n, t = pmf.shape
    bi = 512
    nb = n // bi
    pmfT = pmf.T
    tcol = times.reshape(n, 1)
    trow = times.reshape(1, n)
    ecol = events.astype(jnp.float32).reshape(n, 1)
    tbrow = time_bins.reshape(1, t)
    out = pl.pallas_call(
        _deephit_body,
        grid=(nb,),
        in_specs=[
            pl.BlockSpec((n, t), lambda k: (0, 0)),
            pl.BlockSpec((t, n), lambda k: (0, 0)),
            pl.BlockSpec((n, 1), lambda k: (0, 0)),
            pl.BlockSpec((1, n), lambda k: (0, 0)),
            pl.BlockSpec((n, 1), lambda k: (0, 0)),
            pl.BlockSpec((1, t), lambda k: (0, 0)),
        ],
        out_specs=pl.BlockSpec((1, 1), lambda k: (0, 0)),
        out_shape=jax.ShapeDtypeStruct((1, 1), jnp.float32),
        scratch_shapes=[
            pltpu.VMEM((t, n), jnp.bfloat16),
            pltpu.VMEM((n, 1), jnp.float32),
            pltpu.VMEM((n, 1), jnp.float32),
            pltpu.VMEM((1, 1), jnp.float32),
            pltpu.VMEM((1, 1), jnp.float32),
            pltpu.VMEM((1, 1), jnp.float32),
            pltpu.VMEM((1, 1), jnp.float32),
        ],
    )(pmf, pmfT, tcol, trow, ecol, tbrow)
    return out[0, 0]
